# Initial kernel scaffold; baseline (speedup 1.0000x reference)
import jax
import jax.numpy as jnp
from jax.experimental import pallas as pl


def kernel(h, adj, node_mask, W0, b0, W1, b1, W2, b2, W_out, b_out):
    raise NotImplementedError("write your pallas kernel here")


# speedup vs baseline: 1.1307x; 1.1307x over previous
import jax
import jax.numpy as jnp
from jax.experimental import pallas as pl
from jax.experimental.pallas import tpu as pltpu

_NORM = 100.0


def _gcn_kernel(h_ref, adj_ref, mask_ref, w0_ref, b0_ref, w1_ref, b1_ref,
                w2_ref, b2_ref, wo_ref, bo_ref, out_ref):
    a = adj_ref[0].astype(jnp.bfloat16)
    x = h_ref[0]
    for w_ref, b_ref in ((w0_ref, b0_ref), (w1_ref, b1_ref), (w2_ref, b2_ref)):
        m = jnp.dot(x.astype(jnp.bfloat16), w_ref[...].astype(jnp.bfloat16),
                    preferred_element_type=jnp.float32) + b_ref[...]
        agg = jnp.dot(a, m.astype(jnp.bfloat16),
                      preferred_element_type=jnp.float32) * (1.0 / _NORM)
        x = jnp.maximum(agg, 0.0)
    out = jnp.dot(x.astype(jnp.bfloat16), wo_ref[...].astype(jnp.bfloat16),
                  preferred_element_type=jnp.float32) + bo_ref[...]
    out_ref[0] = out * mask_ref[0]


def kernel(h, adj, node_mask, W0, b0, W1, b1, W2, b2, W_out, b_out):
    B, N, H = h.shape
    F = W_out.shape[1]
    b0r = b0.reshape(1, H)
    b1r = b1.reshape(1, H)
    b2r = b2.reshape(1, H)
    bor = b_out.reshape(1, F)

    full = lambda *shape: pl.BlockSpec(shape, lambda b: (0,) * len(shape))
    per_batch = lambda *shape: pl.BlockSpec((1,) + shape,
                                            lambda b: (b,) + (0,) * len(shape))

    return pl.pallas_call(
        _gcn_kernel,
        grid=(B,),
        in_specs=[
            per_batch(N, H),
            per_batch(N, N),
            per_batch(N, 1),
            full(H, H), full(1, H),
            full(H, H), full(1, H),
            full(H, H), full(1, H),
            full(H, F), full(1, F),
        ],
        out_specs=per_batch(N, F),
        out_shape=jax.ShapeDtypeStruct((B, N, F), jnp.float32),
        compiler_params=pltpu.CompilerParams(
            dimension_semantics=("parallel",),
        ),
    )(h, adj, node_mask, W0, b0r, W1, b1r, W2, b2r, W_out, bor)


# speedup vs baseline: 1.6542x; 1.4630x over previous
import jax
import jax.numpy as jnp
from jax.experimental import pallas as pl
from jax.experimental.pallas import tpu as pltpu

_NORM = 100.0


def _gcn_kernel(h_ref, adj_ref, mask_ref, w0_ref, b0_ref, w1_ref, b1_ref,
                w2_ref, b2_ref, wo_ref, bo_ref, out_ref):
    a = adj_ref[0].astype(jnp.bfloat16)
    x = h_ref[0]
    for w_ref, b_ref in ((w0_ref, b0_ref), (w1_ref, b1_ref), (w2_ref, b2_ref)):
        m = jnp.dot(x.astype(jnp.bfloat16), w_ref[...].astype(jnp.bfloat16),
                    preferred_element_type=jnp.float32) + b_ref[...]
        mb = m.astype(jnp.bfloat16)
        agg = jnp.concatenate(
            [jnp.dot(a[r:r + 256], mb, preferred_element_type=jnp.float32)
             for r in range(0, a.shape[0], 256)],
            axis=0) * (1.0 / _NORM)
        x = jnp.maximum(agg, 0.0)
    out = jnp.dot(x.astype(jnp.bfloat16), wo_ref[...].astype(jnp.bfloat16),
                  preferred_element_type=jnp.float32) + bo_ref[...]
    out_ref[0] = out * mask_ref[0]


def kernel(h, adj, node_mask, W0, b0, W1, b1, W2, b2, W_out, b_out):
    B, N, H = h.shape
    F = W_out.shape[1]
    b0r = b0.reshape(1, H)
    b1r = b1.reshape(1, H)
    b2r = b2.reshape(1, H)
    bor = b_out.reshape(1, F)

    full = lambda *shape: pl.BlockSpec(shape, lambda b: (0,) * len(shape))
    per_batch = lambda *shape: pl.BlockSpec((1,) + shape,
                                            lambda b: (b,) + (0,) * len(shape))

    return pl.pallas_call(
        _gcn_kernel,
        grid=(B,),
        in_specs=[
            per_batch(N, H),
            per_batch(N, N),
            per_batch(N, 1),
            full(H, H), full(1, H),
            full(H, H), full(1, H),
            full(H, H), full(1, H),
            full(H, F), full(1, F),
        ],
        out_specs=per_batch(N, F),
        out_shape=jax.ShapeDtypeStruct((B, N, F), jnp.float32),
        compiler_params=pltpu.CompilerParams(
            dimension_semantics=("parallel",),
        ),
    )(h, adj, node_mask, W0, b0r, W1, b1r, W2, b2r, W_out, bor)
